# baseline (speedup 1.0000x reference)
import functools

import jax
import jax.numpy as jnp
from jax import lax
from jax.experimental import pallas as pl
from jax.experimental.pallas import tpu as pltpu
from jax.experimental.pallas import tpu_sc as plsc

B = 16384
USER_DIM = 64
GENRE_DIM = 32
OUT_DIM = USER_DIM + GENRE_DIM
UVOC = 1001
GVOC = 21
GPI = 10
PAD = 128
NC, NS, L = 2, 16, 16
NW = NC * NS
BPW = B // NW
CH = 128
NCH = BPW // CH
NG = BPW // L
GIW = BPW * GPI // PAD


def _body(uid_hbm, gid_hbm, utab_hbm, gtab_hbm, out_hbm,
          uidx_v, gidx_v, gtab_v, urows_v, obuf_v, usemA, usemB):
  cid = lax.axis_index("c")
  sid = lax.axis_index("s")
  wid = sid * NC + cid
  base = wid * BPW

  pltpu.sync_copy(uid_hbm.at[wid], uidx_v)
  pltpu.sync_copy(gid_hbm.at[wid], gidx_v)
  pltpu.sync_copy(gtab_hbm, gtab_v)

  for c in range(NCH):
    for t in range(CH // L):
      uidx_v[c, pl.ds(t * L, L)] = uidx_v[c, pl.ds(t * L, L)] + 1

  usems = (usemA, usemB)
  ucopies = [
      pltpu.async_copy(utab_hbm.at[uidx_v.at[c]],
                       urows_v.at[c % 2], usems[c % 2])
      for c in range(2)
  ]

  scale = jnp.float32(1.0 / GPI)
  iota = lax.iota(jnp.int32, L)
  seven = jnp.int32(7)
  mask7f = jnp.int32(PAD - 1)
  ovecs = [jnp.full((L,), USER_DIM + c, jnp.int32) for c in range(GENRE_DIM)]
  cvecs = [jnp.full((L,), c, jnp.int32) for c in range(GENRE_DIM)]

  @functools.partial(plsc.parallel_loop, 0, NG, unroll=2)
  def _grp(t):
    item_rows = iota + t * L
    pos0 = item_rows * GPI
    rowsel = []
    for j in range(GPI):
      pos = pos0 + j
      rowsel.append(plsc.load_gather(
          gidx_v, [lax.shift_right_logical(pos, seven), pos & mask7f]))
    accs = [plsc.load_gather(gtab_v, [rowsel[0], cvecs[c]])
            for c in range(GENRE_DIM)]
    for j in range(1, GPI):
      accs = [a + plsc.load_gather(gtab_v, [rowsel[j], cvecs[c]])
              for c, a in enumerate(accs)]
    for c in range(GENRE_DIM):
      plsc.store_scatter(obuf_v, [item_rows, ovecs[c]], accs[c] * scale)

  for c in range(NCH):
    ucopies[c].wait()

    @functools.partial(plsc.parallel_loop, 0, CH, unroll=4)
    def _fold(i, c=c):
      for h in range(USER_DIM // L):
        obuf_v[c * CH + i, pl.ds(h * L, L)] = urows_v[c % 2, i, pl.ds(h * L, L)]

    if c + 2 < NCH:
      ucopies.append(
          pltpu.async_copy(utab_hbm.at[uidx_v.at[c + 2]],
                           urows_v.at[c % 2], usems[c % 2]))

  pltpu.sync_copy(obuf_v, out_hbm.at[pl.ds(base, BPW)])


@jax.jit
def kernel(user_id, movie_genres, user_table, genre_table):
  uid3 = user_id.reshape(NW, NCH, CH)
  gid3 = movie_genres.reshape(NW, GIW, PAD)
  utab_p = jnp.concatenate(
      [user_table, jnp.zeros((UVOC, PAD - USER_DIM), jnp.float32)], axis=1)
  gtab_p = jnp.concatenate(
      [genre_table, jnp.zeros((GVOC, PAD - GENRE_DIM), jnp.float32)], axis=1)

  run = pl.kernel(
      _body,
      out_type=jax.ShapeDtypeStruct((B, OUT_DIM), jnp.float32),
      mesh=plsc.VectorSubcoreMesh(core_axis_name="c", subcore_axis_name="s",
                                  num_cores=NC, num_subcores=NS),
      scratch_types=[
          pltpu.VMEM((NCH, CH), jnp.int32),
          pltpu.VMEM((GIW, PAD), jnp.int32),
          pltpu.VMEM((GVOC, PAD), jnp.float32),
          pltpu.VMEM((2, CH, PAD), jnp.float32),
          pltpu.VMEM((BPW, OUT_DIM), jnp.float32),
          pltpu.SemaphoreType.DMA,
          pltpu.SemaphoreType.DMA,
      ],
      compiler_params=pltpu.CompilerParams(use_tc_tiling_on_sc=True,
                                           needs_layout_passes=False),
  )
  return run(uid3, gid3, utab_p, gtab_p)
